# Initial kernel scaffold; baseline (speedup 1.0000x reference)
import jax
import jax.numpy as jnp
from jax.experimental import pallas as pl


def kernel(x, table):
    raise NotImplementedError("write your pallas kernel here")


# speedup vs baseline: 3.3395x; 3.3395x over previous
import functools

import jax
import jax.numpy as jnp
from jax import lax
from jax.experimental import pallas as pl
from jax.experimental.pallas import tpu as pltpu
from jax.experimental.pallas import tpu_sc as plsc

D_MODEL = 128
CHUNK = 128


def _chunks_per_worker(n_rows: int) -> int:
  info = plsc.get_sparse_core_info()
  return n_rows // (info.num_cores * info.num_subcores) // CHUNK


@functools.cache
def _build(n_rows: int, vocab: int, d: int):
  info = plsc.get_sparse_core_info()
  nc, ns = info.num_cores, info.num_subcores
  nw = nc * ns
  per_w = n_rows // nw
  chunks = per_w // CHUNK
  steps = chunks // 2
  assert n_rows == nw * chunks * CHUNK and chunks % 2 == 0

  mesh = plsc.VectorSubcoreMesh(core_axis_name="c", subcore_axis_name="s")

  def body(idx_hbm, table_hbm, out_hbm, idx_v, buf0, buf1, sem0, sem1):
    wid = lax.axis_index("s") * nc + lax.axis_index("c")
    obase = wid * per_w

    pltpu.sync_copy(idx_hbm.at[wid], idx_v)

    pltpu.async_copy(table_hbm.at[idx_v.at[0]], buf0, sem0)
    pltpu.async_copy(table_hbm.at[idx_v.at[1]], buf1, sem1)

    def wait(sem, buf):
      pltpu.make_async_copy(table_hbm.at[pl.ds(0, CHUNK)], buf, sem).wait()

    def step(c, carry):
      c0 = 2 * c

      wait(sem0, buf0)
      pltpu.sync_copy(buf0, out_hbm.at[pl.ds(obase + c0 * CHUNK, CHUNK)])

      @pl.when(c < steps - 1)
      def _():
        pltpu.async_copy(table_hbm.at[idx_v.at[c0 + 2]], buf0, sem0)

      wait(sem1, buf1)
      pltpu.sync_copy(buf1, out_hbm.at[pl.ds(obase + (c0 + 1) * CHUNK, CHUNK)])

      @pl.when(c < steps - 1)
      def _():
        pltpu.async_copy(table_hbm.at[idx_v.at[c0 + 3]], buf1, sem1)

      return carry

    lax.fori_loop(0, steps, step, 0)

  return pl.kernel(
      body,
      out_type=jax.ShapeDtypeStruct((n_rows, d), jnp.float32),
      mesh=mesh,
      scratch_types=[
          pltpu.VMEM((chunks, CHUNK), jnp.int32),
          pltpu.VMEM((CHUNK, d), jnp.float32),
          pltpu.VMEM((CHUNK, d), jnp.float32),
          pltpu.SemaphoreType.DMA,
          pltpu.SemaphoreType.DMA,
      ],
  )


@jax.jit
def kernel(x, table):
  b, h = x.shape
  vocab, d = table.shape
  n_rows = b * h
  nw = n_rows // CHUNK // _chunks_per_worker(n_rows)
  idx = x.reshape(nw, _chunks_per_worker(n_rows), CHUNK).astype(jnp.int32)
  out = _build(n_rows, vocab, d)(idx, table)
  return out.reshape(b, h, d)


# speedup vs baseline: 3.3545x; 1.0045x over previous
import functools

import jax
import jax.numpy as jnp
from jax import lax
from jax.experimental import pallas as pl
from jax.experimental.pallas import tpu as pltpu
from jax.experimental.pallas import tpu_sc as plsc

D_MODEL = 128
CHUNK = 128


def _chunks_per_worker(n_rows: int) -> int:
  info = plsc.get_sparse_core_info()
  return n_rows // (info.num_cores * info.num_subcores) // CHUNK


@functools.cache
def _build(n_rows: int, vocab: int, d: int):
  info = plsc.get_sparse_core_info()
  nc, ns = info.num_cores, info.num_subcores
  nw = nc * ns
  per_w = n_rows // nw
  chunks = per_w // CHUNK
  nbuf = 5
  steps = chunks // nbuf
  assert n_rows == nw * chunks * CHUNK and chunks % nbuf == 0

  mesh = plsc.VectorSubcoreMesh(core_axis_name="c", subcore_axis_name="s")

  def body(idx_hbm, table_hbm, out_hbm, idx_v, bufs, sems):
    wid = lax.axis_index("s") * nc + lax.axis_index("c")
    obase = wid * per_w

    pltpu.sync_copy(idx_hbm.at[wid], idx_v)

    for b in range(nbuf):
      pltpu.async_copy(table_hbm.at[idx_v.at[b]], bufs[b], sems[b])

    def wait(sem, buf):
      pltpu.make_async_copy(table_hbm.at[pl.ds(0, CHUNK)], buf, sem).wait()

    def step(i, carry):
      c0 = nbuf * i
      for b in range(nbuf):
        wait(sems[b], bufs[b])
        pltpu.sync_copy(bufs[b], out_hbm.at[pl.ds(obase + (c0 + b) * CHUNK, CHUNK)])

        @pl.when(i < steps - 1)
        def _(b=b):
          pltpu.async_copy(table_hbm.at[idx_v.at[c0 + nbuf + b]], bufs[b], sems[b])

      return carry

    lax.fori_loop(0, steps, step, 0)

  return pl.kernel(
      body,
      out_type=jax.ShapeDtypeStruct((n_rows, d), jnp.float32),
      mesh=mesh,
      scratch_types=[
          pltpu.VMEM((chunks, CHUNK), jnp.int32),
          [pltpu.VMEM((CHUNK, d), jnp.float32) for _ in range(nbuf)],
          [pltpu.SemaphoreType.DMA for _ in range(nbuf)],
      ],
  )


@jax.jit
def kernel(x, table):
  b, h = x.shape
  vocab, d = table.shape
  n_rows = b * h
  nw = n_rows // CHUNK // _chunks_per_worker(n_rows)
  idx = x.reshape(nw, _chunks_per_worker(n_rows), CHUNK).astype(jnp.int32)
  out = _build(n_rows, vocab, d)(idx, table)
  return out.reshape(b, h, d)


# speedup vs baseline: 5.9699x; 1.7797x over previous
import functools

import jax
import jax.numpy as jnp
from jax import lax
from jax.experimental import pallas as pl
from jax.experimental.pallas import tpu as pltpu
from jax.experimental.pallas import tpu_sc as plsc


@functools.cache
def _build(batch: int, hist: int, vocab: int, d: int):
  info = plsc.get_sparse_core_info()
  nc, ns = info.num_cores, info.num_subcores
  nw = nc * ns
  per_w = batch // nw
  nbuf = 4
  steps = per_w // nbuf
  assert batch == nw * per_w and per_w % nbuf == 0

  mesh = plsc.VectorSubcoreMesh(core_axis_name="c", subcore_axis_name="s")

  def body(idx_hbm, table_hbm, out_hbm, idx_v, bufs, sems):
    wid = lax.axis_index("s") * nc + lax.axis_index("c")
    obase = wid * per_w

    pltpu.sync_copy(idx_hbm.at[pl.ds(obase, per_w)], idx_v)

    for b in range(nbuf):
      pltpu.async_copy(table_hbm.at[idx_v.at[b]], bufs[b], sems[b])

    def wait(sem, buf):
      pltpu.make_async_copy(table_hbm.at[idx_v.at[0]], buf, sem).wait()

    def step(i, carry):
      j0 = nbuf * i
      for b in range(nbuf):
        wait(sems[b], bufs[b])
        pltpu.sync_copy(bufs[b], out_hbm.at[j0 + b + obase])

        @pl.when(i < steps - 1)
        def _(b=b):
          pltpu.async_copy(table_hbm.at[idx_v.at[j0 + nbuf + b]], bufs[b], sems[b])

      return carry

    lax.fori_loop(0, steps, step, 0)

  return pl.kernel(
      body,
      out_type=jax.ShapeDtypeStruct((batch, hist, d), jnp.float32),
      mesh=mesh,
      scratch_types=[
          pltpu.VMEM((per_w, hist), jnp.int32),
          [pltpu.VMEM((hist, d), jnp.float32) for _ in range(nbuf)],
          [pltpu.SemaphoreType.DMA for _ in range(nbuf)],
      ],
  )


@jax.jit
def kernel(x, table):
  b, h = x.shape
  vocab, d = table.shape
  return _build(b, h, vocab, d)(x.astype(jnp.int32), table)
